# baseline (speedup 1.0000x reference)
import functools

import jax
import jax.numpy as jnp
from jax import lax
from jax.experimental import pallas as pl
from jax.experimental.pallas import tpu as pltpu
from jax.experimental.pallas import tpu_sc as plsc

_VOCAB = 100000
_DIM = 64
_BATCH = 16384
_MARGIN = 1.0

_NW = 32
_PAIRS = 2 * _BATCH
_PER_W = _PAIRS // _NW
_CHUNK = 128
_NCHUNK = _PER_W // _CHUNK
_NGRP = _CHUNK // 16


def _sc_gather_body(idx_h, emb_h, out_h, idxbuf, rows0, rows1, sem0, sem1):
    c = lax.axis_index("c")
    s = lax.axis_index("s")
    wid = c * 16 + s
    pltpu.sync_copy(idx_h.at[wid], idxbuf)
    bufs = (rows0, rows1)
    sems = (sem0, sem1)
    cps = [pltpu.async_copy(emb_h.at[idxbuf.at[0]], rows0, sem0), None]
    for g in range(_NCHUNK):
        cur = g % 2
        nxt = (g + 1) % 2
        if g + 1 < _NCHUNK:
            cps[nxt] = pltpu.async_copy(
                emb_h.at[idxbuf.at[g + 1]], bufs[nxt], sems[nxt])
        cps[cur].wait()
        pltpu.sync_copy(bufs[cur], out_h.at[wid * _NCHUNK + g])


@functools.partial(
    pl.kernel,
    out_type=jax.ShapeDtypeStruct((_NW * _NCHUNK, _CHUNK, _DIM), jnp.float32),
    mesh=plsc.VectorSubcoreMesh(core_axis_name="c", subcore_axis_name="s"),
    compiler_params=pltpu.CompilerParams(
        needs_layout_passes=False, use_tc_tiling_on_sc=False),
    scratch_types=[
        pltpu.VMEM((_NCHUNK, _CHUNK), jnp.int32),
        pltpu.VMEM((_CHUNK, _DIM), jnp.float32),
        pltpu.VMEM((_CHUNK, _DIM), jnp.float32),
        pltpu.SemaphoreType.DMA,
        pltpu.SemaphoreType.DMA,
    ],
)
def _sc_gather(idx, emb, out, idxbuf, rows0, rows1, sem0, sem1):
    _sc_gather_body(idx, emb, out, idxbuf, rows0, rows1, sem0, sem1)


def _sc_body(conc_idx_h, gi_h, conc_emb_h, out_h,
             idx_c, rows_i, rows_c, pbuf, accv, sem):
    c = lax.axis_index("c")
    s = lax.axis_index("s")
    wid = c * 16 + s
    sign = jnp.where(wid < 16, -1.0, 1.0).astype(jnp.float32)

    pltpu.sync_copy(conc_idx_h.at[wid], idx_c)

    iota16 = lax.iota(jnp.int32, 16)
    tbase = iota16 * 16

    def chunk_body(g, acc):
        cp1 = pltpu.async_copy(gi_h.at[wid * _NCHUNK + g], rows_i, sem)
        cp2 = pltpu.async_copy(conc_emb_h.at[idx_c.at[g]], rows_c, sem)
        cp1.wait()
        cp2.wait()

        def grp_body(grp, acc):
            for k in range(16):
                p = grp * 16 + k
                partial = jnp.zeros((16,), jnp.float32)
                for d in range(_DIM // 16):
                    vi = rows_i[p, pl.ds(d * 16, 16)]
                    vc = rows_c[p, pl.ds(d * 16, 16)]
                    partial = partial + vi * vc
                pbuf[pl.ds(k * 16, 16)] = partial
            f = jnp.zeros((16,), jnp.float32)
            for l in range(16):
                f = f + plsc.load_gather(pbuf, [tbase + l])
            return acc + jnp.maximum(sign * (f - _MARGIN), 0.0)

        return lax.fori_loop(0, _NGRP, grp_body, acc)

    acc = lax.fori_loop(0, _NCHUNK, chunk_body, jnp.zeros((16,), jnp.float32))
    accv[...] = acc
    pltpu.sync_copy(accv, out_h.at[wid])


@functools.partial(
    pl.kernel,
    out_type=jax.ShapeDtypeStruct((_NW, 16), jnp.float32),
    mesh=plsc.VectorSubcoreMesh(core_axis_name="c", subcore_axis_name="s"),
    compiler_params=pltpu.CompilerParams(
        needs_layout_passes=False, use_tc_tiling_on_sc=False),
    scratch_types=[
        pltpu.VMEM((_NCHUNK, _CHUNK), jnp.int32),
        pltpu.VMEM((_CHUNK, _DIM), jnp.float32),
        pltpu.VMEM((_CHUNK, _DIM), jnp.float32),
        pltpu.VMEM((256,), jnp.float32),
        pltpu.VMEM((16,), jnp.float32),
        pltpu.SemaphoreType.DMA,
    ],
)
def _sc_loss(conc_idx, gi, conc_emb, out, idx_c, rows_i, rows_c, pbuf, accv, sem):
    _sc_body(conc_idx, gi, conc_emb, out, idx_c, rows_i, rows_c, pbuf, accv, sem)


_TCOLS = 8192
_TGRID = (_VOCAB + _TCOLS - 1) // _TCOLS
_VOCAB_PAD = _TGRID * _TCOLS


@functools.partial(
    pl.pallas_call,
    grid=(_TGRID,),
    in_specs=[pl.BlockSpec((_DIM, _TCOLS), lambda j: (0, j))],
    out_specs=pl.BlockSpec((_TCOLS // 2, 128), lambda j: (j, 0)),
    out_shape=jax.ShapeDtypeStruct((_VOCAB_PAD // 2, 128), jnp.float32),
)
def _tc_repack(t_ref, o_ref):
    y = t_ref[...].T
    h = _TCOLS // 2
    o_ref[...] = jnp.concatenate([y[:h, :], y[h:, :]], axis=1)


def _remap_rows(r):
    return ((r >> 13) << 13) + ((r & 4095) << 1) + ((r >> 12) & 1)


def kernel(batch_pos, batch_neg, instance_emb, concept_emb):
    inst_idx = _remap_rows(jnp.concatenate(
        [batch_pos[:, 0], batch_neg[:, 0]]).astype(jnp.int32)).reshape(
            _NW, _NCHUNK, _CHUNK)
    conc_idx = _remap_rows(jnp.concatenate(
        [batch_pos[:, 1], batch_neg[:, 1]]).astype(jnp.int32)).reshape(
            _NW, _NCHUNK, _CHUNK)
    inst_lin = _tc_repack(instance_emb.astype(jnp.float32).T)
    gi = _sc_gather(inst_idx, inst_lin.reshape(_VOCAB_PAD, _DIM))
    conc_lin = _tc_repack(concept_emb.astype(jnp.float32).T)
    partials = _sc_loss(conc_idx, gi, conc_lin.reshape(_VOCAB_PAD, _DIM))
    return partials.sum()


# speedup vs baseline: 1.1607x; 1.1607x over previous
import functools

import jax
import jax.numpy as jnp
from jax import lax
from jax.experimental import pallas as pl
from jax.experimental.pallas import tpu as pltpu
from jax.experimental.pallas import tpu_sc as plsc

_VOCAB = 100000
_DIM = 64
_BATCH = 16384
_MARGIN = 1.0

_NW = 32
_PAIRS = 2 * _BATCH
_PER_W = _PAIRS // _NW
_CHUNK = 128
_NCHUNK = _PER_W // _CHUNK
_NGRP = _CHUNK // 16


def _sc_body(inst_idx_h, conc_idx_h, inst_emb_h, conc_emb_h, out_h,
             idx_i, idx_c, rows_i0, rows_c0, rows_i1, rows_c1,
             pbuf, accv, sem0, sem1):
    c = lax.axis_index("c")
    s = lax.axis_index("s")
    wid = c * 16 + s
    sign = jnp.where(wid < 16, -1.0, 1.0).astype(jnp.float32)

    pltpu.sync_copy(inst_idx_h.at[wid], idx_i)
    pltpu.sync_copy(conc_idx_h.at[wid], idx_c)

    iota16 = lax.iota(jnp.int32, 16)
    tbase = iota16 * 16

    bufs = ((rows_i0, rows_c0), (rows_i1, rows_c1))
    sems = (sem0, sem1)

    def start_gather(g, b):
        return (pltpu.async_copy(inst_emb_h.at[idx_i.at[g]], bufs[b][0], sems[b]),
                pltpu.async_copy(conc_emb_h.at[idx_c.at[g]], bufs[b][1], sems[b]))

    cps = [start_gather(0, 0), None]
    acc = jnp.zeros((16,), jnp.float32)
    for g in range(_NCHUNK):
        cur = g % 2
        nxt = (g + 1) % 2
        if g + 1 < _NCHUNK:
            cps[nxt] = start_gather(g + 1, nxt)
        cps[cur][0].wait()
        cps[cur][1].wait()
        rows_i, rows_c = bufs[cur]

        def grp_body(grp, acc, rows_i=rows_i, rows_c=rows_c):
            for k in range(16):
                p = grp * 16 + k
                partial = jnp.zeros((16,), jnp.float32)
                for d in range(_DIM // 16):
                    vi = rows_i[p, pl.ds(d * 16, 16)]
                    vc = rows_c[p, pl.ds(d * 16, 16)]
                    partial = partial + vi * vc
                pbuf[pl.ds(k * 16, 16)] = partial
            f = jnp.zeros((16,), jnp.float32)
            for l in range(16):
                f = f + plsc.load_gather(pbuf, [tbase + l])
            return acc + jnp.maximum(sign * (f - _MARGIN), 0.0)

        acc = lax.fori_loop(0, _NGRP, grp_body, acc)

    accv[...] = acc
    pltpu.sync_copy(accv, out_h.at[wid])


@functools.partial(
    pl.kernel,
    out_type=jax.ShapeDtypeStruct((_NW, 16), jnp.float32),
    mesh=plsc.VectorSubcoreMesh(core_axis_name="c", subcore_axis_name="s"),
    compiler_params=pltpu.CompilerParams(
        needs_layout_passes=False, use_tc_tiling_on_sc=False),
    scratch_types=[
        pltpu.VMEM((_NCHUNK, _CHUNK), jnp.int32),
        pltpu.VMEM((_NCHUNK, _CHUNK), jnp.int32),
        pltpu.VMEM((_CHUNK, _DIM), jnp.float32),
        pltpu.VMEM((_CHUNK, _DIM), jnp.float32),
        pltpu.VMEM((_CHUNK, _DIM), jnp.float32),
        pltpu.VMEM((_CHUNK, _DIM), jnp.float32),
        pltpu.VMEM((256,), jnp.float32),
        pltpu.VMEM((16,), jnp.float32),
        pltpu.SemaphoreType.DMA,
        pltpu.SemaphoreType.DMA,
    ],
)
def _sc_loss(inst_idx, conc_idx, inst_emb, conc_emb, out,
             idx_i, idx_c, rows_i0, rows_c0, rows_i1, rows_c1,
             pbuf, accv, sem0, sem1):
    _sc_body(inst_idx, conc_idx, inst_emb, conc_emb, out,
             idx_i, idx_c, rows_i0, rows_c0, rows_i1, rows_c1,
             pbuf, accv, sem0, sem1)


_TCOLS = 8192
_TGRID = (_VOCAB + _TCOLS - 1) // _TCOLS
_VOCAB_PAD = _TGRID * _TCOLS


@functools.partial(
    pl.pallas_call,
    grid=(_TGRID,),
    in_specs=[
        pl.BlockSpec((_DIM, _TCOLS), lambda j: (0, j)),
        pl.BlockSpec((_DIM, _TCOLS), lambda j: (0, j)),
    ],
    out_specs=[
        pl.BlockSpec((_TCOLS // 2, 128), lambda j: (j, 0)),
        pl.BlockSpec((_TCOLS // 2, 128), lambda j: (j, 0)),
    ],
    out_shape=[
        jax.ShapeDtypeStruct((_VOCAB_PAD // 2, 128), jnp.float32),
        jax.ShapeDtypeStruct((_VOCAB_PAD // 2, 128), jnp.float32),
    ],
)
def _tc_repack(ti_ref, tc_ref, oi_ref, oc_ref):
    yi = ti_ref[...].T
    yc = tc_ref[...].T
    h = _TCOLS // 2
    oi_ref[...] = jnp.concatenate([yi[:h, :], yi[h:, :]], axis=1)
    oc_ref[...] = jnp.concatenate([yc[:h, :], yc[h:, :]], axis=1)


def _remap_rows(r):
    return ((r >> 13) << 13) + ((r & 4095) << 1) + ((r >> 12) & 1)


def kernel(batch_pos, batch_neg, instance_emb, concept_emb):
    inst_idx = _remap_rows(jnp.concatenate(
        [batch_pos[:, 0], batch_neg[:, 0]]).astype(jnp.int32)).reshape(
            _NW, _NCHUNK, _CHUNK)
    conc_idx = _remap_rows(jnp.concatenate(
        [batch_pos[:, 1], batch_neg[:, 1]]).astype(jnp.int32)).reshape(
            _NW, _NCHUNK, _CHUNK)
    inst_lin, conc_lin = _tc_repack(
        instance_emb.astype(jnp.float32).T, concept_emb.astype(jnp.float32).T)
    partials = _sc_loss(inst_idx, conc_idx,
                        inst_lin.reshape(_VOCAB_PAD, _DIM),
                        conc_lin.reshape(_VOCAB_PAD, _DIM))
    return partials.sum()
